# baseline (speedup 1.0000x reference)
import functools

import jax
import jax.numpy as jnp
from jax import lax
from jax.experimental import pallas as pl
from jax.experimental.pallas import tpu as pltpu
from jax.experimental.pallas import tpu_sc as plsc

D = 128
L = 16
NC, NS = 2, 16
NW = NC * NS
N = 10000
E = 160000
EPT = E // NW
CH_NC = 128
CH_C = 112
NCH_NC = 40
NCH_C = 45
NPAD = 10240
ROWS_PT = NPAD // NS


def _sc_pass_body(with_c, *refs):
    if with_c:
        (p_hbm, q_hbm, idx_hbm, c_hbm, u_hbm, out_hbm,
         acc, iv, pv, qv, cv, uv, sem1, sem2) = refs
    else:
        (p_hbm, q_hbm, idx_hbm, u_hbm, out_hbm,
         acc, iv, pv, qv, uv, sem1, sem2) = refs
        cv = None
    ch = CH_C if with_c else CH_NC
    nchunk = NCH_C if with_c else NCH_NC
    cid = lax.axis_index("c")
    sid = lax.axis_index("s")
    wid = cid * NS + sid
    zero = jnp.zeros((L,), jnp.float32)

    pltpu.sync_copy(u_hbm, uv)

    def zrow(e, carry):
        for j in range(D // L):
            pv[e, pl.ds(j * L, L)] = zero
        return carry
    lax.fori_loop(0, ch, zrow, 0)

    base_row = sid * ROWS_PT
    off = 0
    while off < ROWS_PT:
        rn = min(ch, ROWS_PT - off)
        pltpu.sync_copy(pv.at[pl.ds(0, rn), :], acc.at[pl.ds(base_row + off, rn), :])
        off += rn
    plsc.subcore_barrier()

    us = tuple(uv[j] for j in range(D // L))

    def chunk(k, carry):
        pltpu.sync_copy(idx_hbm.at[wid, k], iv)
        cp1 = pltpu.async_copy(p_hbm.at[iv.at[0]], pv, sem1)
        cp2 = pltpu.async_copy(q_hbm.at[iv.at[8]], qv, sem2)
        if with_c:
            pltpu.sync_copy(c_hbm.at[wid, pl.ds(k * ch, ch), :], cv)
        cp1.wait()
        cp2.wait()

        def edge(e, carry2):
            for j in range(D // L):
                s = pl.ds(j * L, L)
                v = pv[e, s] + qv[e, s]
                if with_c:
                    v = v + cv[e, s]
                pv[e, s] = jnp.maximum(v, jnp.float32(0.0)) + us[j]
            return carry2
        lax.fori_loop(0, ch, edge, 0)
        pltpu.sync_copy(pv, acc.at[iv.at[8]], add=True)
        return carry
    lax.fori_loop(0, nchunk, chunk, 0)
    plsc.subcore_barrier()
    pltpu.sync_copy(acc.at[pl.ds(base_row, ROWS_PT), :],
                    out_hbm.at[cid, pl.ds(base_row, ROWS_PT), :])


@functools.lru_cache(maxsize=None)
def _make_sc_pass(with_c):
    mesh = plsc.VectorSubcoreMesh(core_axis_name="c", subcore_axis_name="s",
                                  num_cores=NC, num_subcores=NS)
    ch = CH_C if with_c else CH_NC
    scratch = [
        pltpu.VMEM_SHARED((NPAD, D), jnp.float32),
        pltpu.VMEM((16, ch), jnp.int32),
        pltpu.VMEM((ch, D), jnp.float32),
        pltpu.VMEM((ch, D), jnp.float32),
    ]
    if with_c:
        scratch.append(pltpu.VMEM((ch, D), jnp.float32))
    scratch += [
        pltpu.VMEM((D // L, L), jnp.float32),
        pltpu.SemaphoreType.DMA,
        pltpu.SemaphoreType.DMA,
    ]
    return pl.kernel(
        functools.partial(_sc_pass_body, with_c),
        out_type=jax.ShapeDtypeStruct((NC, NPAD, D), jnp.float32),
        mesh=mesh,
        scratch_types=scratch,
    )


def _pad_edges(src, dst, ch, nchunk):
    src = src.reshape(NW, EPT)
    dst = dst.reshape(NW, EPT)
    padn = nchunk * ch - EPT
    src = jnp.pad(src, ((0, 0), (0, padn))).reshape(NW, nchunk, ch)
    dst = jnp.pad(dst, ((0, 0), (0, padn)),
                  constant_values=NPAD - 1).reshape(NW, nchunk, ch)
    idx = jnp.zeros((NW, nchunk, 16, ch), jnp.int32)
    idx = idx.at[:, :, 0, :].set(src).at[:, :, 8, :].set(dst)
    return idx


def _pad_q(q):
    return jnp.pad(q, ((0, NPAD - N), (0, 0)))


def _sc_pass(p, q, src, dst, u):
    idx = _pad_edges(src, dst, CH_NC, NCH_NC)
    return _make_sc_pass(False)(p, _pad_q(q), idx, u)[:, :N]


def _sc_pass_c(p, q, src, dst, c, u):
    idx = _pad_edges(src, dst, CH_C, NCH_C)
    return _make_sc_pass(True)(p, _pad_q(q), idx, c, u)[:, :N]


def _projpair_body(x1_ref, x2_ref, wt_ref, wb_ref, b_ref, a_ref, bo_ref):
    a_ref[...] = jnp.dot(x1_ref[...], wt_ref[...],
                         preferred_element_type=jnp.float32)
    bo_ref[...] = jnp.dot(x2_ref[...], wb_ref[...],
                          preferred_element_type=jnp.float32) + b_ref[...]


def _projpair(x1, x2, wtop, wbot, b):
    R = 1000
    return pl.pallas_call(
        _projpair_body,
        grid=(N // R,),
        in_specs=[
            pl.BlockSpec((R, D), lambda i: (i, 0)),
            pl.BlockSpec((R, D), lambda i: (i, 0)),
            pl.BlockSpec((D, D), lambda i: (0, 0)),
            pl.BlockSpec((D, D), lambda i: (0, 0)),
            pl.BlockSpec((1, D), lambda i: (0, 0)),
        ],
        out_specs=[pl.BlockSpec((R, D), lambda i: (i, 0)),
                   pl.BlockSpec((R, D), lambda i: (i, 0))],
        out_shape=[jax.ShapeDtypeStruct((N, D), jnp.float32)] * 2,
    )(x1, x2, wtop, wbot, b.reshape(1, D))


def _mm16_body(x_ref, w_ref, o_ref):
    o_ref[...] = jnp.dot(x_ref[...], w_ref[...],
                         preferred_element_type=jnp.float32)


def _mm16(x, w):
    R = 512
    rows, k = x.shape
    assert rows % R == 0
    return pl.pallas_call(
        _mm16_body,
        grid=(rows // R,),
        in_specs=[
            pl.BlockSpec((R, k), lambda i: (i, 0)),
            pl.BlockSpec((k, D), lambda i: (0, 0)),
        ],
        out_specs=pl.BlockSpec((R, D), lambda i: (i, 0)),
        out_shape=jax.ShapeDtypeStruct((rows, D), jnp.float32),
    )(x, w)


def _combine_ln_body(nk, s_ref, w_ref, xres_ref, g_ref, b_ref, o_ref):
    acc = xres_ref[...]
    for k in range(nk):
        acc = acc + jnp.dot(s_ref[k], w_ref[k],
                            preferred_element_type=jnp.float32)
    m = jnp.mean(acc, axis=-1, keepdims=True)
    v = jnp.mean((acc - m) ** 2, axis=-1, keepdims=True)
    o_ref[...] = (acc - m) / jnp.sqrt(v + 1e-5) * g_ref[...] + b_ref[...]


def _combine_ln(s_stack, w_stack, xres, g, b):
    nk = s_stack.shape[0]
    R = 1000
    return pl.pallas_call(
        functools.partial(_combine_ln_body, nk),
        grid=(N // R,),
        in_specs=[
            pl.BlockSpec((nk, R, D), lambda i: (0, i, 0)),
            pl.BlockSpec((nk, D, D), lambda i: (0, 0, 0)),
            pl.BlockSpec((R, D), lambda i: (i, 0)),
            pl.BlockSpec((1, D), lambda i: (0, 0)),
            pl.BlockSpec((1, D), lambda i: (0, 0)),
        ],
        out_specs=pl.BlockSpec((R, D), lambda i: (i, 0)),
        out_shape=jax.ShapeDtypeStruct((N, D), jnp.float32),
    )(s_stack, w_stack, xres, g.reshape(1, D), b.reshape(1, D))


def _bias_u(w2, b2):
    return jnp.linalg.solve(w2.T, b2).reshape(D // L, L)


def kernel(x_st, x_scalar, x_vector, adj_edge_index, adj_edge_attr, inh_scalar_edge_index, inh_vector_edge_index, inv_scalar_edge_index, inv_vector_edge_index, f2s_scalar_W1, f2s_scalar_b1, f2s_scalar_W2, f2s_scalar_b2, f2s_vector_W1, f2s_vector_b1, f2s_vector_W2, f2s_vector_b2, st2st_W1, st2st_b1, st2st_W2, st2st_b2, s2f_scalar_W1, s2f_scalar_b1, s2f_scalar_W2, s2f_scalar_b2, s2f_vector_W1, s2f_vector_b1, s2f_vector_W2, s2f_vector_b2, st_norm_g, st_norm_b, norm_scalar_g, norm_scalar_b, norm_vector_g, norm_vector_b):
    a1s, b1s = _projpair(x_scalar, x_st, f2s_scalar_W1[:D], f2s_scalar_W1[D:],
                         f2s_scalar_b1)
    a1v, b1v = _projpair(x_vector, x_st, f2s_vector_W1[:D], f2s_vector_W1[D:],
                         f2s_vector_b1)
    s1s = _sc_pass(a1s, b1s, inh_scalar_edge_index[0], inh_scalar_edge_index[1],
                   _bias_u(f2s_scalar_W2, f2s_scalar_b2))
    s1v = _sc_pass(a1v, b1v, inh_vector_edge_index[0], inh_vector_edge_index[1],
                   _bias_u(f2s_vector_W2, f2s_vector_b2))
    x = _combine_ln(jnp.concatenate([s1s, s1v], axis=0),
                    jnp.stack([f2s_scalar_W2, f2s_scalar_W2,
                               f2s_vector_W2, f2s_vector_W2]),
                    x_st, st_norm_g, st_norm_b)

    a2, b2p = _projpair(x, x, st2st_W1[:D], st2st_W1[D:2 * D], st2st_b1)
    ec = NCH_C * CH_C
    ke = adj_edge_attr.shape[1]
    attr_p = jnp.pad(adj_edge_attr.reshape(NW, EPT, ke),
                     ((0, 0), (0, ec - EPT), (0, 0)))
    c2 = _mm16(attr_p.reshape(NW * ec, ke), st2st_W1[2 * D:]).reshape(NW, ec, D)
    s2 = _sc_pass_c(a2, b2p, adj_edge_index[0], adj_edge_index[1], c2,
                    _bias_u(st2st_W2, st2st_b2))
    x2 = _combine_ln(s2, jnp.stack([st2st_W2, st2st_W2]), x,
                     st_norm_g, st_norm_b)

    outs = []
    for xf, ei, w1, bb1, w2, bb2, g, b in (
        (x_scalar, inv_scalar_edge_index, s2f_scalar_W1, s2f_scalar_b1,
         s2f_scalar_W2, s2f_scalar_b2, norm_scalar_g, norm_scalar_b),
        (x_vector, inv_vector_edge_index, s2f_vector_W1, s2f_vector_b1,
         s2f_vector_W2, s2f_vector_b2, norm_vector_g, norm_vector_b),
    ):
        a3, b3 = _projpair(x2, xf, w1[:D], w1[D:], bb1)
        s3 = _sc_pass(a3, b3, ei[0], ei[1], _bias_u(w2, bb2))
        outs.append(_combine_ln(s3, jnp.stack([w2, w2]), xf, g, b))
    return (x2, outs[0], outs[1])


# speedup vs baseline: 1.0048x; 1.0048x over previous
import functools

import jax
import jax.numpy as jnp
from jax import lax
from jax.experimental import pallas as pl
from jax.experimental.pallas import tpu as pltpu
from jax.experimental.pallas import tpu_sc as plsc

D = 128
L = 16
NC, NS = 2, 16
NW = NC * NS
N = 10000
E = 160000
EPT = E // NW
CH_NC = 128
CH_C = 112
NCH_NC = 40
NCH_C = 46
NPAD = 10240
ROWS_PT = NPAD // NS


def _sc_pass_body(with_c, *refs):
    if with_c:
        (p_hbm, q_hbm, src_hbm, dst_hbm, c_hbm, u_hbm, out_hbm,
         acc, sv0, sv1, dv0, dv1, pv, qv, cv, uv, smi0, smi1, sem1, sem2) = refs
    else:
        (p_hbm, q_hbm, src_hbm, dst_hbm, u_hbm, out_hbm,
         acc, sv0, sv1, dv0, dv1, pv, qv, uv, smi0, smi1, sem1, sem2) = refs
        cv = None
    ch = CH_C if with_c else CH_NC
    nchunk = NCH_C if with_c else NCH_NC
    svs, dvs = (sv0, sv1), (dv0, dv1)
    smis = (smi0, smi1)
    cid = lax.axis_index("c")
    sid = lax.axis_index("s")
    wid = cid * NS + sid
    zero = jnp.zeros((L,), jnp.float32)

    pltpu.sync_copy(u_hbm, uv)

    def zrow(e, carry):
        for j in range(D // L):
            pv[e, pl.ds(j * L, L)] = zero
        return carry
    lax.fori_loop(0, ch, zrow, 0)

    base_row = sid * ROWS_PT
    off = 0
    while off < ROWS_PT:
        rn = min(ch, ROWS_PT - off)
        pltpu.sync_copy(pv.at[pl.ds(0, rn), :], acc.at[pl.ds(base_row + off, rn), :])
        off += rn
    plsc.subcore_barrier()

    us = tuple(uv[j] for j in range(D // L))

    def issue_idx(k, s):
        pltpu.async_copy(src_hbm.at[wid, k], svs[s], smis[s])
        pltpu.async_copy(dst_hbm.at[wid, k], dvs[s], smis[s])

    def wait_idx(k, s):
        pltpu.make_async_copy(src_hbm.at[wid, k], svs[s], smis[s]).wait()
        pltpu.make_async_copy(dst_hbm.at[wid, k], dvs[s], smis[s]).wait()

    issue_idx(0, 0)

    def superstep(g, carry):
        for s in (0, 1):
            k = 2 * g + s

            @pl.when(k + 1 < nchunk)
            def _():
                issue_idx(k + 1, 1 - s)
            wait_idx(k, s)
            cp1 = pltpu.async_copy(p_hbm.at[svs[s]], pv, sem1)
            cp2 = pltpu.async_copy(q_hbm.at[dvs[s]], qv, sem2)
            if with_c:
                pltpu.sync_copy(c_hbm.at[wid, pl.ds(k * ch, ch), :], cv)
            cp1.wait()
            cp2.wait()

            def edge(e, carry2):
                for j in range(D // L):
                    sl = pl.ds(j * L, L)
                    v = pv[e, sl] + qv[e, sl]
                    if with_c:
                        v = v + cv[e, sl]
                    pv[e, sl] = jnp.maximum(v, jnp.float32(0.0)) + us[j]
                return carry2
            lax.fori_loop(0, ch, edge, 0)
            pltpu.sync_copy(pv, acc.at[dvs[s]], add=True)
        return carry
    lax.fori_loop(0, nchunk // 2, superstep, 0)
    plsc.subcore_barrier()
    pltpu.sync_copy(acc.at[pl.ds(base_row, ROWS_PT), :],
                    out_hbm.at[cid, pl.ds(base_row, ROWS_PT), :])


@functools.lru_cache(maxsize=None)
def _make_sc_pass(with_c):
    mesh = plsc.VectorSubcoreMesh(core_axis_name="c", subcore_axis_name="s",
                                  num_cores=NC, num_subcores=NS)
    ch = CH_C if with_c else CH_NC
    scratch = [
        pltpu.VMEM_SHARED((NPAD, D), jnp.float32),
        pltpu.VMEM((ch,), jnp.int32),
        pltpu.VMEM((ch,), jnp.int32),
        pltpu.VMEM((ch,), jnp.int32),
        pltpu.VMEM((ch,), jnp.int32),
        pltpu.VMEM((ch, D), jnp.float32),
        pltpu.VMEM((ch, D), jnp.float32),
    ]
    if with_c:
        scratch.append(pltpu.VMEM((ch, D), jnp.float32))
    scratch += [
        pltpu.VMEM((D // L, L), jnp.float32),
        pltpu.SemaphoreType.DMA,
        pltpu.SemaphoreType.DMA,
        pltpu.SemaphoreType.DMA,
        pltpu.SemaphoreType.DMA,
    ]
    return pl.kernel(
        functools.partial(_sc_pass_body, with_c),
        out_type=jax.ShapeDtypeStruct((NC, NPAD, D), jnp.float32),
        mesh=mesh,
        scratch_types=scratch,
    )


def _pad_edges(src, dst, ch, nchunk):
    src = src.reshape(NW, EPT)
    dst = dst.reshape(NW, EPT)
    padn = nchunk * ch - EPT
    src = jnp.pad(src, ((0, 0), (0, padn))).reshape(NW, nchunk, ch)
    dst = jnp.pad(dst, ((0, 0), (0, padn)),
                  constant_values=NPAD - 1).reshape(NW, nchunk, ch)
    return src, dst


def _pad_q(q):
    return jnp.pad(q, ((0, NPAD - N), (0, 0)))


def _sc_pass(p, q, src, dst, u):
    s, d = _pad_edges(src, dst, CH_NC, NCH_NC)
    return _make_sc_pass(False)(p, _pad_q(q), s, d, u)[:, :N]


def _sc_pass_c(p, q, src, dst, c, u):
    s, d = _pad_edges(src, dst, CH_C, NCH_C)
    return _make_sc_pass(True)(p, _pad_q(q), s, d, c, u)[:, :N]


def _projpair_body(x1_ref, x2_ref, wt_ref, wb_ref, b_ref, a_ref, bo_ref):
    a_ref[...] = jnp.dot(x1_ref[...], wt_ref[...],
                         preferred_element_type=jnp.float32)
    bo_ref[...] = jnp.dot(x2_ref[...], wb_ref[...],
                          preferred_element_type=jnp.float32) + b_ref[...]


def _projpair(x1, x2, wtop, wbot, b):
    R = 1000
    return pl.pallas_call(
        _projpair_body,
        grid=(N // R,),
        in_specs=[
            pl.BlockSpec((R, D), lambda i: (i, 0)),
            pl.BlockSpec((R, D), lambda i: (i, 0)),
            pl.BlockSpec((D, D), lambda i: (0, 0)),
            pl.BlockSpec((D, D), lambda i: (0, 0)),
            pl.BlockSpec((1, D), lambda i: (0, 0)),
        ],
        out_specs=[pl.BlockSpec((R, D), lambda i: (i, 0)),
                   pl.BlockSpec((R, D), lambda i: (i, 0))],
        out_shape=[jax.ShapeDtypeStruct((N, D), jnp.float32)] * 2,
    )(x1, x2, wtop, wbot, b.reshape(1, D))


def _mm16_body(x_ref, w_ref, o_ref):
    o_ref[...] = jnp.dot(x_ref[...], w_ref[...],
                         preferred_element_type=jnp.float32)


def _mm16(x, w):
    R = 512
    rows, k = x.shape
    assert rows % R == 0
    return pl.pallas_call(
        _mm16_body,
        grid=(rows // R,),
        in_specs=[
            pl.BlockSpec((R, k), lambda i: (i, 0)),
            pl.BlockSpec((k, D), lambda i: (0, 0)),
        ],
        out_specs=pl.BlockSpec((R, D), lambda i: (i, 0)),
        out_shape=jax.ShapeDtypeStruct((rows, D), jnp.float32),
    )(x, w)


def _combine_ln_body(nk, s_ref, w_ref, xres_ref, g_ref, b_ref, o_ref):
    acc = xres_ref[...]
    for k in range(nk):
        acc = acc + jnp.dot(s_ref[k], w_ref[k],
                            preferred_element_type=jnp.float32)
    m = jnp.mean(acc, axis=-1, keepdims=True)
    v = jnp.mean((acc - m) ** 2, axis=-1, keepdims=True)
    o_ref[...] = (acc - m) / jnp.sqrt(v + 1e-5) * g_ref[...] + b_ref[...]


def _combine_ln(s_stack, w_stack, xres, g, b):
    nk = s_stack.shape[0]
    R = 1000
    return pl.pallas_call(
        functools.partial(_combine_ln_body, nk),
        grid=(N // R,),
        in_specs=[
            pl.BlockSpec((nk, R, D), lambda i: (0, i, 0)),
            pl.BlockSpec((nk, D, D), lambda i: (0, 0, 0)),
            pl.BlockSpec((R, D), lambda i: (i, 0)),
            pl.BlockSpec((1, D), lambda i: (0, 0)),
            pl.BlockSpec((1, D), lambda i: (0, 0)),
        ],
        out_specs=pl.BlockSpec((R, D), lambda i: (i, 0)),
        out_shape=jax.ShapeDtypeStruct((N, D), jnp.float32),
    )(s_stack, w_stack, xres, g.reshape(1, D), b.reshape(1, D))


def _bias_u(w2, b2):
    return jnp.linalg.solve(w2.T, b2).reshape(D // L, L)


def kernel(x_st, x_scalar, x_vector, adj_edge_index, adj_edge_attr, inh_scalar_edge_index, inh_vector_edge_index, inv_scalar_edge_index, inv_vector_edge_index, f2s_scalar_W1, f2s_scalar_b1, f2s_scalar_W2, f2s_scalar_b2, f2s_vector_W1, f2s_vector_b1, f2s_vector_W2, f2s_vector_b2, st2st_W1, st2st_b1, st2st_W2, st2st_b2, s2f_scalar_W1, s2f_scalar_b1, s2f_scalar_W2, s2f_scalar_b2, s2f_vector_W1, s2f_vector_b1, s2f_vector_W2, s2f_vector_b2, st_norm_g, st_norm_b, norm_scalar_g, norm_scalar_b, norm_vector_g, norm_vector_b):
    a1s, b1s = _projpair(x_scalar, x_st, f2s_scalar_W1[:D], f2s_scalar_W1[D:],
                         f2s_scalar_b1)
    a1v, b1v = _projpair(x_vector, x_st, f2s_vector_W1[:D], f2s_vector_W1[D:],
                         f2s_vector_b1)
    s1s = _sc_pass(a1s, b1s, inh_scalar_edge_index[0], inh_scalar_edge_index[1],
                   _bias_u(f2s_scalar_W2, f2s_scalar_b2))
    s1v = _sc_pass(a1v, b1v, inh_vector_edge_index[0], inh_vector_edge_index[1],
                   _bias_u(f2s_vector_W2, f2s_vector_b2))
    x = _combine_ln(jnp.concatenate([s1s, s1v], axis=0),
                    jnp.stack([f2s_scalar_W2, f2s_scalar_W2,
                               f2s_vector_W2, f2s_vector_W2]),
                    x_st, st_norm_g, st_norm_b)

    a2, b2p = _projpair(x, x, st2st_W1[:D], st2st_W1[D:2 * D], st2st_b1)
    ec = NCH_C * CH_C
    ke = adj_edge_attr.shape[1]
    attr_p = jnp.pad(adj_edge_attr.reshape(NW, EPT, ke),
                     ((0, 0), (0, ec - EPT), (0, 0)))
    c2 = _mm16(attr_p.reshape(NW * ec, ke), st2st_W1[2 * D:]).reshape(NW, ec, D)
    s2 = _sc_pass_c(a2, b2p, adj_edge_index[0], adj_edge_index[1], c2,
                    _bias_u(st2st_W2, st2st_b2))
    x2 = _combine_ln(s2, jnp.stack([st2st_W2, st2st_W2]), x,
                     st_norm_g, st_norm_b)

    outs = []
    for xf, ei, w1, bb1, w2, bb2, g, b in (
        (x_scalar, inv_scalar_edge_index, s2f_scalar_W1, s2f_scalar_b1,
         s2f_scalar_W2, s2f_scalar_b2, norm_scalar_g, norm_scalar_b),
        (x_vector, inv_vector_edge_index, s2f_vector_W1, s2f_vector_b1,
         s2f_vector_W2, s2f_vector_b2, norm_vector_g, norm_vector_b),
    ):
        a3, b3 = _projpair(x2, xf, w1[:D], w1[D:], bb1)
        s3 = _sc_pass(a3, b3, ei[0], ei[1], _bias_u(w2, bb2))
        outs.append(_combine_ln(s3, jnp.stack([w2, w2]), xf, g, b))
    return (x2, outs[0], outs[1])


# speedup vs baseline: 1.4586x; 1.4516x over previous
import functools

import jax
import jax.numpy as jnp
from jax import lax
from jax.experimental import pallas as pl
from jax.experimental.pallas import tpu as pltpu
from jax.experimental.pallas import tpu_sc as plsc

D = 128
L = 16
NC, NS = 2, 16
NW = NC * NS
N = 10000
E = 160000
EPT = E // NW
CH_NC = 128
CH_C = 120
NPAD = 10240
ROWS_PT = NPAD // NS


def _sc_pass_body(with_c, *refs):
    if with_c:
        (p_hbm, q_hbm, src_hbm, dst_hbm, c_hbm, u_hbm, out_hbm,
         acc, srcv, dstv, srcv_t, dstv_t, pv, qv, cv, uv, sem1, sem2) = refs
    else:
        (p_hbm, q_hbm, src_hbm, dst_hbm, u_hbm, out_hbm,
         acc, srcv, dstv, srcv_t, dstv_t, pv, qv, uv, sem1, sem2) = refs
        cv = None
    ch = CH_C if with_c else CH_NC
    nfull = EPT // ch
    tail = EPT - nfull * ch
    cid = lax.axis_index("c")
    sid = lax.axis_index("s")
    zero = jnp.zeros((L,), jnp.float32)

    pltpu.sync_copy(u_hbm, uv)

    def zrow(e, carry):
        for j in range(D // L):
            pv[e, pl.ds(j * L, L)] = zero
        return carry
    lax.fori_loop(0, ch, zrow, 0)

    base_row = sid * ROWS_PT
    off = 0
    while off < ROWS_PT:
        rn = min(ch, ROWS_PT - off)
        pltpu.sync_copy(pv.at[pl.ds(0, rn), :], acc.at[pl.ds(base_row + off, rn), :])
        off += rn
    plsc.subcore_barrier()

    us = tuple(uv[j] for j in range(D // L))
    ebase = (cid * NS + sid) * EPT

    def do_edges(b, n, sv, dv):
        pltpu.sync_copy(src_hbm.at[pl.ds(b, n)], sv)
        pltpu.sync_copy(dst_hbm.at[pl.ds(b, n)], dv)
        cp1 = pltpu.async_copy(p_hbm.at[sv], pv.at[pl.ds(0, n), :], sem1)
        cp2 = pltpu.async_copy(q_hbm.at[dv], qv.at[pl.ds(0, n), :], sem2)
        if with_c:
            pltpu.sync_copy(c_hbm.at[pl.ds(b, n), :], cv.at[pl.ds(0, n), :])
        cp1.wait()
        cp2.wait()

        def edge(e, carry2):
            for j in range(D // L):
                s = pl.ds(j * L, L)
                v = pv[e, s] + qv[e, s]
                if with_c:
                    v = v + cv[e, s]
                pv[e, s] = jnp.maximum(v, jnp.float32(0.0)) + us[j]
            return carry2
        lax.fori_loop(0, n, edge, 0)
        pltpu.sync_copy(pv.at[pl.ds(0, n), :], acc.at[dv], add=True)

    def chunk(k, carry):
        do_edges(ebase + k * ch, ch, srcv, dstv)
        return carry
    lax.fori_loop(0, nfull, chunk, 0)
    if tail:
        do_edges(ebase + nfull * ch, tail, srcv_t, dstv_t)
    plsc.subcore_barrier()
    pltpu.sync_copy(acc.at[pl.ds(base_row, ROWS_PT), :],
                    out_hbm.at[cid, pl.ds(base_row, ROWS_PT), :])


@functools.lru_cache(maxsize=None)
def _make_sc_pass(with_c):
    mesh = plsc.VectorSubcoreMesh(core_axis_name="c", subcore_axis_name="s",
                                  num_cores=NC, num_subcores=NS)
    ch = CH_C if with_c else CH_NC
    tail = EPT - (EPT // ch) * ch
    scratch = [
        pltpu.VMEM_SHARED((NPAD, D), jnp.float32),
        pltpu.VMEM((ch,), jnp.int32),
        pltpu.VMEM((ch,), jnp.int32),
        pltpu.VMEM((max(tail, 8),), jnp.int32),
        pltpu.VMEM((max(tail, 8),), jnp.int32),
        pltpu.VMEM((ch, D), jnp.float32),
        pltpu.VMEM((ch, D), jnp.float32),
    ]
    if with_c:
        scratch.append(pltpu.VMEM((ch, D), jnp.float32))
    scratch += [
        pltpu.VMEM((D // L, L), jnp.float32),
        pltpu.SemaphoreType.DMA,
        pltpu.SemaphoreType.DMA,
    ]
    return pl.kernel(
        functools.partial(_sc_pass_body, with_c),
        out_type=jax.ShapeDtypeStruct((NC, NPAD, D), jnp.float32),
        mesh=mesh,
        scratch_types=scratch,
    )


def _sc_pass(p, q, src, dst, u):
    return _make_sc_pass(False)(p, q, src, dst, u)[:, :N]


def _sc_pass_c(p, q, src, dst, c, u):
    return _make_sc_pass(True)(p, q, src, dst, c, u)[:, :N]


def _projpair_body(x1_ref, x2_ref, wt_ref, wb_ref, b_ref, a_ref, bo_ref):
    a_ref[...] = jnp.dot(x1_ref[...], wt_ref[...],
                         preferred_element_type=jnp.float32)
    bo_ref[...] = jnp.dot(x2_ref[...], wb_ref[...],
                          preferred_element_type=jnp.float32) + b_ref[...]


def _projpair(x1, x2, wtop, wbot, b):
    R = 1000
    return pl.pallas_call(
        _projpair_body,
        grid=(N // R,),
        in_specs=[
            pl.BlockSpec((R, D), lambda i: (i, 0)),
            pl.BlockSpec((R, D), lambda i: (i, 0)),
            pl.BlockSpec((D, D), lambda i: (0, 0)),
            pl.BlockSpec((D, D), lambda i: (0, 0)),
            pl.BlockSpec((1, D), lambda i: (0, 0)),
        ],
        out_specs=[pl.BlockSpec((R, D), lambda i: (i, 0)),
                   pl.BlockSpec((R, D), lambda i: (i, 0))],
        out_shape=[jax.ShapeDtypeStruct((N, D), jnp.float32)] * 2,
    )(x1, x2, wtop, wbot, b.reshape(1, D))


def _mm16_body(x_ref, w_ref, o_ref):
    o_ref[...] = jnp.dot(x_ref[...], w_ref[...],
                         preferred_element_type=jnp.float32)


def _mm16(x, w):
    R = 640
    rows, k = x.shape
    assert rows % R == 0
    return pl.pallas_call(
        _mm16_body,
        grid=(rows // R,),
        in_specs=[
            pl.BlockSpec((R, k), lambda i: (i, 0)),
            pl.BlockSpec((k, D), lambda i: (0, 0)),
        ],
        out_specs=pl.BlockSpec((R, D), lambda i: (i, 0)),
        out_shape=jax.ShapeDtypeStruct((rows, D), jnp.float32),
    )(x, w)


def _combine_ln_body(nk, s_ref, w_ref, xres_ref, g_ref, b_ref, o_ref):
    acc = xres_ref[...]
    for k in range(nk):
        acc = acc + jnp.dot(s_ref[k], w_ref[k],
                            preferred_element_type=jnp.float32)
    m = jnp.mean(acc, axis=-1, keepdims=True)
    v = jnp.mean((acc - m) ** 2, axis=-1, keepdims=True)
    o_ref[...] = (acc - m) / jnp.sqrt(v + 1e-5) * g_ref[...] + b_ref[...]


def _combine_ln(s_stack, w_stack, xres, g, b):
    nk = s_stack.shape[0]
    R = 1000
    return pl.pallas_call(
        functools.partial(_combine_ln_body, nk),
        grid=(N // R,),
        in_specs=[
            pl.BlockSpec((nk, R, D), lambda i: (0, i, 0)),
            pl.BlockSpec((nk, D, D), lambda i: (0, 0, 0)),
            pl.BlockSpec((R, D), lambda i: (i, 0)),
            pl.BlockSpec((1, D), lambda i: (0, 0)),
            pl.BlockSpec((1, D), lambda i: (0, 0)),
        ],
        out_specs=pl.BlockSpec((R, D), lambda i: (i, 0)),
        out_shape=jax.ShapeDtypeStruct((N, D), jnp.float32),
    )(s_stack, w_stack, xres, g.reshape(1, D), b.reshape(1, D))


def _bias_u(w2, b2):
    return jnp.linalg.solve(w2.T, b2).reshape(D // L, L)


def kernel(x_st, x_scalar, x_vector, adj_edge_index, adj_edge_attr, inh_scalar_edge_index, inh_vector_edge_index, inv_scalar_edge_index, inv_vector_edge_index, f2s_scalar_W1, f2s_scalar_b1, f2s_scalar_W2, f2s_scalar_b2, f2s_vector_W1, f2s_vector_b1, f2s_vector_W2, f2s_vector_b2, st2st_W1, st2st_b1, st2st_W2, st2st_b2, s2f_scalar_W1, s2f_scalar_b1, s2f_scalar_W2, s2f_scalar_b2, s2f_vector_W1, s2f_vector_b1, s2f_vector_W2, s2f_vector_b2, st_norm_g, st_norm_b, norm_scalar_g, norm_scalar_b, norm_vector_g, norm_vector_b):
    a1s, b1s = _projpair(x_scalar, x_st, f2s_scalar_W1[:D], f2s_scalar_W1[D:],
                         f2s_scalar_b1)
    a1v, b1v = _projpair(x_vector, x_st, f2s_vector_W1[:D], f2s_vector_W1[D:],
                         f2s_vector_b1)
    s1s = _sc_pass(a1s, b1s, inh_scalar_edge_index[0], inh_scalar_edge_index[1],
                   _bias_u(f2s_scalar_W2, f2s_scalar_b2))
    s1v = _sc_pass(a1v, b1v, inh_vector_edge_index[0], inh_vector_edge_index[1],
                   _bias_u(f2s_vector_W2, f2s_vector_b2))
    x = _combine_ln(jnp.concatenate([s1s, s1v], axis=0),
                    jnp.stack([f2s_scalar_W2, f2s_scalar_W2,
                               f2s_vector_W2, f2s_vector_W2]),
                    x_st, st_norm_g, st_norm_b)

    a2, b2p = _projpair(x, x, st2st_W1[:D], st2st_W1[D:2 * D], st2st_b1)
    c2 = _mm16(adj_edge_attr, st2st_W1[2 * D:])
    s2 = _sc_pass_c(a2, b2p, adj_edge_index[0], adj_edge_index[1], c2,
                    _bias_u(st2st_W2, st2st_b2))
    x2 = _combine_ln(s2, jnp.stack([st2st_W2, st2st_W2]), x,
                     st_norm_g, st_norm_b)

    outs = []
    for xf, ei, w1, bb1, w2, bb2, g, b in (
        (x_scalar, inv_scalar_edge_index, s2f_scalar_W1, s2f_scalar_b1,
         s2f_scalar_W2, s2f_scalar_b2, norm_scalar_g, norm_scalar_b),
        (x_vector, inv_vector_edge_index, s2f_vector_W1, s2f_vector_b1,
         s2f_vector_W2, s2f_vector_b2, norm_vector_g, norm_vector_b),
    ):
        a3, b3 = _projpair(x2, xf, w1[:D], w1[D:], bb1)
        s3 = _sc_pass(a3, b3, ei[0], ei[1], _bias_u(w2, bb2))
        outs.append(_combine_ln(s3, jnp.stack([w2, w2]), xf, g, b))
    return (x2, outs[0], outs[1])
